# baseline (speedup 1.0000x reference)
import functools

import jax
import jax.numpy as jnp
from jax import lax
from jax.experimental import pallas as pl
from jax.experimental.pallas import tpu as pltpu
from jax.experimental.pallas import tpu_sc as plsc

N_NODES = 10000
N_EDGES = 320000
D_FEAT = 128
D_EDGE = 16
DZ = 256

NC = 2
NS = 16
NW = NC * NS
EPT = N_EDGES // NW
CHUNK = 40
BLK_CHUNKS = 10
BLK = CHUNK * BLK_CHUNKS
NBLK = EPT // BLK
ROWS_PER_TILE = 624
ACC_ROWS = 10240


def _node_tables_body(x_ref, wp_ref, wq_ref, p_ref, q_ref):
    xb = x_ref[...]
    p_ref[...] = jnp.dot(xb, wp_ref[...], preferred_element_type=jnp.float32)
    q_ref[...] = jnp.dot(xb, wq_ref[...], preferred_element_type=jnp.float32)


def _node_tables(x, wp, wq):
    blk = 1000
    grid = N_NODES // blk
    return pl.pallas_call(
        _node_tables_body,
        grid=(grid,),
        in_specs=[
            pl.BlockSpec((blk, D_FEAT), lambda i: (i, 0)),
            pl.BlockSpec((D_FEAT, DZ), lambda i: (0, 0)),
            pl.BlockSpec((D_FEAT, DZ), lambda i: (0, 0)),
        ],
        out_specs=[
            pl.BlockSpec((blk, DZ), lambda i: (i, 0)),
            pl.BlockSpec((blk, DZ), lambda i: (i, 0)),
        ],
        out_shape=[
            jax.ShapeDtypeStruct((N_NODES, DZ), jnp.float32),
            jax.ShapeDtypeStruct((N_NODES, DZ), jnp.float32),
        ],
    )(x, wp, wq)


def _edge_table_body(ea_ref, we_ref, b_ref, e_ref):
    e_ref[...] = (
        jnp.dot(ea_ref[...], we_ref[...], preferred_element_type=jnp.float32)
        + b_ref[...]
    )


def _edge_table(ea, we, bcat):
    blk = 2000
    grid = N_EDGES // blk
    return pl.pallas_call(
        _edge_table_body,
        grid=(grid,),
        in_specs=[
            pl.BlockSpec((blk, D_EDGE), lambda i: (i, 0)),
            pl.BlockSpec((D_EDGE, DZ), lambda i: (0, 0)),
            pl.BlockSpec((1, DZ), lambda i: (0, 0)),
        ],
        out_specs=pl.BlockSpec((blk, DZ), lambda i: (i, 0)),
        out_shape=jax.ShapeDtypeStruct((N_EDGES, DZ), jnp.float32),
    )(ea, we, bcat)


def _softplus(zs):
    a = jnp.abs(zs)
    en = jnp.exp(-a)
    t = en / (en + 2.0)
    t2 = t * t
    poly = ((2.0 / 9.0 * t2 + 2.0 / 7.0) * t2 + 2.0 / 5.0) * t2 + 2.0 / 3.0
    lnm = t * (poly * t2 + 2.0)
    return jnp.maximum(zs, 0.0) + lnm


def _sc_body(p_hbm, q_hbm, e_hbm, dst_hbm, src_hbm, dist_hbm, out_hbm,
             dst_blk, src_blk, wblk, dstbuf, srcbuf, pbuf, qbuf, ebuf, mbuf,
             acc, sem):
    cid = lax.axis_index("c")
    sid = lax.axis_index("s")
    wid = cid * NS + sid
    ebase = wid * EPT

    def _zero_m(i, _):
        for f in range(8):
            mbuf[i, pl.ds(16 * f, 16)] = jnp.zeros((16,), jnp.float32)
        return 0
    lax.fori_loop(0, CHUNK, _zero_m, 0)
    for j in range(16):
        pltpu.sync_copy(mbuf, acc.at[pl.ds(sid * 640 + j * CHUNK, CHUNK)])

    plsc.subcore_barrier()

    def _block(b, _):
        bbase = ebase + b * BLK
        cp_d = pltpu.async_copy(dst_hbm.at[pl.ds(bbase, BLK)], dst_blk, sem)
        cp_s = pltpu.async_copy(src_hbm.at[pl.ds(bbase, BLK)], src_blk, sem)
        cp_w = pltpu.async_copy(dist_hbm.at[pl.ds(bbase, BLK)], wblk, sem)
        cp_d.wait()
        cp_s.wait()
        cp_w.wait()

        def _wgt(i, _):
            d = wblk[pl.ds(16 * i, 16)]
            wblk[pl.ds(16 * i, 16)] = jnp.exp(d * d * (-1.0 / 18.0))
            return 0
        lax.fori_loop(0, BLK // 16, _wgt, 0)

        def _chunk(c, _):
            base = c * CHUNK
            for k in range(CHUNK // 8):
                dstbuf[pl.ds(8 * k, 8)] = dst_blk[pl.ds(base + 8 * k, 8)]
                srcbuf[pl.ds(8 * k, 8)] = src_blk[pl.ds(base + 8 * k, 8)]
            cp_p = pltpu.async_copy(p_hbm.at[dstbuf], pbuf, sem)
            cp_q = pltpu.async_copy(q_hbm.at[srcbuf], qbuf, sem)
            cp_e = pltpu.async_copy(
                e_hbm.at[pl.ds(bbase + base, CHUNK)], ebuf, sem)
            cp_p.wait()
            cp_q.wait()
            cp_e.wait()

            def _edge(e, _):
                boff = base + e
                g = boff // 16
                lane = boff - g * 16
                w16 = wblk[pl.ds(g * 16, 16)]
                wsp = lax.gather(
                    w16, jnp.full((16, 1), lane, jnp.int32),
                    lax.GatherDimensionNumbers(
                        offset_dims=(), collapsed_slice_dims=(0,),
                        start_index_map=(0,)),
                    slice_sizes=(1,),
                    mode=lax.GatherScatterMode.PROMISE_IN_BOUNDS)
                for f in range(8):
                    lo = 16 * f
                    hi = 128 + 16 * f
                    zf = (pbuf[e, pl.ds(lo, 16)] + qbuf[e, pl.ds(lo, 16)]
                          + ebuf[e, pl.ds(lo, 16)])
                    zs = (pbuf[e, pl.ds(hi, 16)] + qbuf[e, pl.ds(hi, 16)]
                          + ebuf[e, pl.ds(hi, 16)])
                    sig = 1.0 / (1.0 + jnp.exp(-zf))
                    mbuf[e, pl.ds(lo, 16)] = sig * _softplus(zs) * wsp
                return 0
            lax.fori_loop(0, CHUNK, _edge, 0)

            pltpu.sync_copy(mbuf, acc.at[dstbuf], add=True)
            return 0
        lax.fori_loop(0, BLK_CHUNKS, _chunk, 0)
        return 0
    lax.fori_loop(0, NBLK, _block, 0)

    plsc.subcore_barrier()

    pltpu.sync_copy(
        acc.at[pl.ds(sid * ROWS_PER_TILE, ROWS_PER_TILE)],
        out_hbm.at[cid, pl.ds(sid * ROWS_PER_TILE, ROWS_PER_TILE)],
    )
    tail = NS * ROWS_PER_TILE

    @pl.when(sid == 0)
    def _copy_tail():
        pltpu.sync_copy(
            acc.at[pl.ds(tail, N_NODES - tail)],
            out_hbm.at[cid, pl.ds(tail, N_NODES - tail)],
        )


def _sc_aggregate(p, q, ecat, dst, src, dist):
    mesh = plsc.VectorSubcoreMesh(core_axis_name="c", subcore_axis_name="s")
    return pl.kernel(
        _sc_body,
        out_type=jax.ShapeDtypeStruct((NC, N_NODES, D_FEAT), jnp.float32),
        mesh=mesh,
        scratch_types=[
            pltpu.VMEM((BLK,), jnp.int32),
            pltpu.VMEM((BLK,), jnp.int32),
            pltpu.VMEM((BLK,), jnp.float32),
            pltpu.VMEM((CHUNK,), jnp.int32),
            pltpu.VMEM((CHUNK,), jnp.int32),
            pltpu.VMEM((CHUNK, DZ), jnp.float32),
            pltpu.VMEM((CHUNK, DZ), jnp.float32),
            pltpu.VMEM((CHUNK, DZ), jnp.float32),
            pltpu.VMEM((CHUNK, D_FEAT), jnp.float32),
            pltpu.VMEM_SHARED((ACC_ROWS, D_FEAT), jnp.float32),
            pltpu.SemaphoreType.DMA,
        ],
    )(p, q, ecat, dst, src, dist)


def _combine_body(part_ref, x_ref, o_ref):
    o_ref[...] = part_ref[0] + part_ref[1] + x_ref[...]


def _combine(partial, x):
    blk = 1000
    grid = N_NODES // blk
    return pl.pallas_call(
        _combine_body,
        grid=(grid,),
        in_specs=[
            pl.BlockSpec((NC, blk, D_FEAT), lambda i: (0, i, 0)),
            pl.BlockSpec((blk, D_FEAT), lambda i: (i, 0)),
        ],
        out_specs=pl.BlockSpec((blk, D_FEAT), lambda i: (i, 0)),
        out_shape=jax.ShapeDtypeStruct((N_NODES, D_FEAT), jnp.float32),
    )(partial, x)


def kernel(x, edge_index, edge_attr, batch, distance, W_f, b_f, W_s, b_s):
    del batch
    F = D_FEAT
    wp = jnp.concatenate([W_f[:, :F].T, W_s[:, :F].T], axis=1)
    wq = jnp.concatenate([W_f[:, F:2 * F].T, W_s[:, F:2 * F].T], axis=1)
    we = jnp.concatenate([W_f[:, 2 * F:].T, W_s[:, 2 * F:].T], axis=1)
    bcat = jnp.concatenate([b_f, b_s]).reshape(1, DZ)

    src = edge_index[0].astype(jnp.int32)
    dst = edge_index[1].astype(jnp.int32)

    p, q = _node_tables(x, wp, wq)
    ecat = _edge_table(edge_attr, we, bcat)
    partial = _sc_aggregate(p, q, ecat, dst, src, distance)
    return _combine(partial, x)
